# baseline (speedup 1.0000x reference)
import jax
import jax.numpy as jnp
from jax.experimental import pallas as pl
from jax.experimental.pallas import tpu as pltpu

_BR = 400
_BR2 = 400


def _prep_kernel(a_ref, x_ref, w1t_ref, b1_ref, dis_ref, g1_ref, ab_ref):
    a = a_ref[...]
    deg = jnp.sum(a, axis=1, keepdims=True) + 1.0
    dis = jnp.where(deg > 0, jax.lax.rsqrt(deg), 0.0)
    h1 = (
        jnp.dot(x_ref[...], w1t_ref[...], preferred_element_type=jnp.float32)
        + b1_ref[...]
    )
    dis_ref[...] = dis
    g1_ref[...] = (dis * h1).astype(jnp.bfloat16)
    ab_ref[...] = pltpu.bitcast(a.astype(jnp.bfloat16), jnp.uint32)


_NS = 1
_HB = _BR2 // _NS


def _mid_kernel(*refs):
    ab_refs = refs[:_NS]
    g1_ref, dis_ref, w2t_ref, b2_ref, g2_ref = refs[_NS:]
    i = pl.program_id(0)
    g1 = g1_ref[...]
    dis = dis_ref[...]
    for h, ab_ref in enumerate(ab_refs):
        ab = pltpu.bitcast(ab_ref[...], jnp.bfloat16)
        acc = jnp.dot(ab, g1, preferred_element_type=jnp.float32)
        g1_i = g1_ref[pl.ds(i * _BR2 + h * _HB, _HB), :].astype(jnp.float32)
        dis_h = dis[h * _HB:(h + 1) * _HB, :]
        out1 = jnp.maximum(dis_h * (acc + g1_i), 0.0)
        h2 = (
            jnp.dot(out1, w2t_ref[...], preferred_element_type=jnp.float32)
            + b2_ref[...]
        )
        g2_ref[h * _HB:(h + 1) * _HB, :] = (dis_h * h2).astype(jnp.bfloat16)


def _final_kernel(*refs):
    ab_refs = refs[:_NS]
    g2_ref, dis_ref, out_ref = refs[_NS:]
    i = pl.program_id(0)
    g2 = g2_ref[...]
    dis = dis_ref[...]
    for h, ab_ref in enumerate(ab_refs):
        ab = pltpu.bitcast(ab_ref[...], jnp.bfloat16)
        acc = jnp.dot(ab, g2, preferred_element_type=jnp.float32)
        g2_i = g2_ref[pl.ds(i * _BR2 + h * _HB, _HB), :].astype(jnp.float32)
        dis_h = dis[h * _HB:(h + 1) * _HB, :]
        out_ref[h * _HB:(h + 1) * _HB, :] = jnp.maximum(
            dis_h * (acc + g2_i), 0.0
        )


@jax.jit
def kernel(x, graph_structure, W1, b1, W2, b2):
    n, d_in = x.shape
    hid = W1.shape[0]
    out_dim = W2.shape[0]
    a = graph_structure
    w1t = W1.T
    w2t = W2.T
    b1r = b1.reshape(1, hid)
    b2r = b2.reshape(1, out_dim)
    grid = (n // _BR,)

    a_spec = pl.BlockSpec((_BR, n), lambda i: (i, 0))
    a2_specs = [
        pl.BlockSpec((_HB // 2, n), lambda i, k=k: (_NS * i + k, 0))
        for k in range(_NS)
    ]
    row_vec = lambda w: pl.BlockSpec((_BR, w), lambda i: (i, 0))
    row_vec2 = lambda w: pl.BlockSpec((_BR2, w), lambda i: (i, 0))
    full = lambda s: pl.BlockSpec(s, lambda i: (0, 0))

    dis, g1, ab = pl.pallas_call(
        _prep_kernel,
        grid=grid,
        in_specs=[a_spec, row_vec(d_in), full((d_in, hid)), full((1, hid))],
        out_specs=[
            row_vec(1),
            row_vec(hid),
            pl.BlockSpec((_BR // 2, n), lambda i: (i, 0)),
        ],
        out_shape=[
            jax.ShapeDtypeStruct((n, 1), jnp.float32),
            jax.ShapeDtypeStruct((n, hid), jnp.bfloat16),
            jax.ShapeDtypeStruct((n // 2, n), jnp.uint32),
        ],
    )(a, x, w1t, b1r)

    grid2 = (n // _BR2,)
    g2 = pl.pallas_call(
        _mid_kernel,
        grid=grid2,
        in_specs=a2_specs
        + [
            full((n, hid)),
            row_vec2(1),
            full((hid, out_dim)),
            full((1, out_dim)),
        ],
        out_specs=row_vec2(out_dim),
        out_shape=jax.ShapeDtypeStruct((n, out_dim), jnp.bfloat16),
    )(*([ab] * _NS), g1, dis, w2t, b2r)

    out = pl.pallas_call(
        _final_kernel,
        grid=grid2,
        in_specs=a2_specs + [full((n, out_dim)), row_vec2(1)],
        out_specs=row_vec2(out_dim),
        out_shape=jax.ShapeDtypeStruct((n, out_dim), jnp.float32),
    )(*([ab] * _NS), g2, dis)

    return out


# speedup vs baseline: 1.0556x; 1.0556x over previous
import jax
import jax.numpy as jnp
from jax.experimental import pallas as pl
from jax.experimental.pallas import tpu as pltpu

_BR = 400
_BR2 = 1000


def _prep_kernel(a_ref, x_ref, w1t_ref, b1_ref, dis_ref, g1_ref, ab_ref):
    a = a_ref[...]
    deg = jnp.sum(a, axis=1, keepdims=True) + 1.0
    dis = jnp.where(deg > 0, jax.lax.rsqrt(deg), 0.0)
    h1 = (
        jnp.dot(x_ref[...], w1t_ref[...], preferred_element_type=jnp.float32)
        + b1_ref[...]
    )
    dis_ref[...] = dis
    g1_ref[...] = (dis * h1).astype(jnp.bfloat16)
    ab_ref[...] = a.astype(jnp.bfloat16)


def _fused23_kernel(
    ab_ref, g1_ref, dis_ref, w2t_ref, b2_ref, out_ref, g2s_ref
):
    i = pl.program_id(0)
    nstr = pl.num_programs(0) // 2
    dis = dis_ref[...]

    @pl.when(i < nstr)
    def _layer1():
        acc = jnp.dot(
            ab_ref[...], g1_ref[...], preferred_element_type=jnp.float32
        )
        g1_i = g1_ref[pl.ds(i * _BR2, _BR2), :].astype(jnp.float32)
        out1 = jnp.maximum(dis * (acc + g1_i), 0.0)
        h2 = (
            jnp.dot(out1, w2t_ref[...], preferred_element_type=jnp.float32)
            + b2_ref[...]
        )
        g2s_ref[pl.ds(i * _BR2, _BR2), :] = (dis * h2).astype(jnp.bfloat16)

    @pl.when(i >= nstr)
    def _layer2():
        j = i - nstr
        acc = jnp.dot(
            ab_ref[...], g2s_ref[...], preferred_element_type=jnp.float32
        )
        g2_i = g2s_ref[pl.ds(j * _BR2, _BR2), :].astype(jnp.float32)
        out_ref[...] = jnp.maximum(dis * (acc + g2_i), 0.0)


@jax.jit
def kernel(x, graph_structure, W1, b1, W2, b2):
    n, d_in = x.shape
    hid = W1.shape[0]
    out_dim = W2.shape[0]
    a = graph_structure
    w1t = W1.T
    w2t = W2.T
    b1r = b1.reshape(1, hid)
    b2r = b2.reshape(1, out_dim)
    grid = (n // _BR,)

    a_spec = pl.BlockSpec((_BR, n), lambda i: (i, 0))
    a2_spec = pl.BlockSpec((_BR2, n), lambda i: (i, 0))
    row_vec = lambda w: pl.BlockSpec((_BR, w), lambda i: (i, 0))
    row_vec2 = lambda w: pl.BlockSpec((_BR2, w), lambda i: (i, 0))
    full = lambda s: pl.BlockSpec(s, lambda i: (0, 0))

    dis, g1, ab = pl.pallas_call(
        _prep_kernel,
        grid=grid,
        in_specs=[a_spec, row_vec(d_in), full((d_in, hid)), full((1, hid))],
        out_specs=[row_vec(1), row_vec(hid), a_spec],
        out_shape=[
            jax.ShapeDtypeStruct((n, 1), jnp.float32),
            jax.ShapeDtypeStruct((n, hid), jnp.bfloat16),
            jax.ShapeDtypeStruct((n, n), jnp.bfloat16),
        ],
    )(a, x, w1t, b1r)

    nstr = n // _BR2
    phase_row = lambda w: pl.BlockSpec(
        (_BR2, w), lambda i: (jnp.where(i < nstr, i, i - nstr), 0)
    )
    out = pl.pallas_call(
        _fused23_kernel,
        grid=(2 * nstr,),
        in_specs=[
            phase_row(n),
            full((n, hid)),
            phase_row(1),
            full((hid, out_dim)),
            full((1, out_dim)),
        ],
        out_specs=pl.BlockSpec(
            (_BR2, out_dim), lambda i: (jnp.maximum(i - nstr, 0), 0)
        ),
        out_shape=jax.ShapeDtypeStruct((n, out_dim), jnp.float32),
        scratch_shapes=[pltpu.VMEM((n, out_dim), jnp.bfloat16)],
    )(ab, g1, dis, w2t, b2r)

    return out


# speedup vs baseline: 1.0569x; 1.0013x over previous
import jax
import jax.numpy as jnp
from jax.experimental import pallas as pl
from jax.experimental.pallas import tpu as pltpu

_BR = 400
_BR2 = 1000


def _prep_kernel(a_ref, x_ref, w1t_ref, b1_ref, dis_ref, g1_ref, ab_ref):
    a = a_ref[...]
    deg = jnp.sum(a, axis=1, keepdims=True) + 1.0
    dis = jnp.where(deg > 0, jax.lax.rsqrt(deg), 0.0)
    h1 = (
        jnp.dot(x_ref[...], w1t_ref[...], preferred_element_type=jnp.float32)
        + b1_ref[...]
    )
    dis_ref[...] = dis
    g1_ref[...] = (dis * h1).astype(jnp.bfloat16)
    ab_ref[...] = a.astype(jnp.bfloat16)


def _fused23_kernel(
    ab_ref, g1_ref, dis_ref, w2t_ref, b2_ref, out_ref, g2s_ref
):
    i = pl.program_id(0)
    nstr = pl.num_programs(0) // 2
    dis = dis_ref[...]

    @pl.when(i < nstr)
    def _layer1():
        acc = jnp.dot(
            ab_ref[...], g1_ref[...], preferred_element_type=jnp.float32
        )
        g1_i = g1_ref[pl.ds(i * _BR2, _BR2), :].astype(jnp.float32)
        out1 = jnp.maximum(dis * (acc + g1_i), 0.0)
        h2 = (
            jnp.dot(out1, w2t_ref[...], preferred_element_type=jnp.float32)
            + b2_ref[...]
        )
        g2s_ref[pl.ds(i * _BR2, _BR2), :] = (dis * h2).astype(jnp.bfloat16)

    @pl.when(i >= nstr)
    def _layer2():
        j = i - nstr
        acc = jnp.dot(
            ab_ref[...], g2s_ref[...], preferred_element_type=jnp.float32
        )
        g2_i = g2s_ref[pl.ds(j * _BR2, _BR2), :].astype(jnp.float32)
        out_ref[...] = jnp.maximum(dis * (acc + g2_i), 0.0)


@jax.jit
def kernel(x, graph_structure, W1, b1, W2, b2):
    n, d_in = x.shape
    hid = W1.shape[0]
    out_dim = W2.shape[0]
    a = graph_structure
    w1t = W1.T
    w2t = W2.T
    b1r = b1.reshape(1, hid)
    b2r = b2.reshape(1, out_dim)
    grid = (n // _BR,)

    a_spec = pl.BlockSpec((_BR, n), lambda i: (i, 0))
    row_vec = lambda w: pl.BlockSpec((_BR, w), lambda i: (i, 0))
    full = lambda s: pl.BlockSpec(s, lambda i: (0, 0))

    dis, g1, ab = pl.pallas_call(
        _prep_kernel,
        grid=grid,
        in_specs=[a_spec, row_vec(d_in), full((d_in, hid)), full((1, hid))],
        out_specs=[row_vec(1), row_vec(hid), a_spec],
        out_shape=[
            jax.ShapeDtypeStruct((n, 1), jnp.float32),
            jax.ShapeDtypeStruct((n, hid), jnp.bfloat16),
            jax.ShapeDtypeStruct((n, n), jnp.bfloat16),
        ],
    )(a, x, w1t, b1r)

    nstr = n // _BR2
    phase_row = lambda w: pl.BlockSpec(
        (_BR2, w), lambda i: (jnp.where(i < nstr, i, i - nstr), 0)
    )
    out = pl.pallas_call(
        _fused23_kernel,
        grid=(2 * nstr,),
        in_specs=[
            phase_row(n),
            full((n, hid)),
            phase_row(1),
            full((hid, out_dim)),
            full((1, out_dim)),
        ],
        out_specs=pl.BlockSpec(
            (_BR2, out_dim), lambda i: (jnp.maximum(i - nstr, 0), 0)
        ),
        out_shape=jax.ShapeDtypeStruct((n, out_dim), jnp.float32),
        scratch_shapes=[pltpu.VMEM((n, out_dim), jnp.bfloat16)],
    )(ab, g1, dis, w2t, b2r)

    return out
